# baseline (speedup 1.0000x reference)
import functools

import jax
import jax.numpy as jnp
from jax import lax
from jax.experimental import pallas as pl
from jax.experimental.pallas import tpu as pltpu
from jax.experimental.pallas import tpu_sc as plsc

_THRESHOLD = 0.5

_NC = 2

_ROWS = 2 * 192 * 384
_W = 384
_ROWS_PER_C = _ROWS // _NC
_CH = 2304
_N_CH = _ROWS_PER_C // _CH
_NBUF = 2
_PD = 1


def _sc_copy(x_hbm, o_hbm, bufs, in_sems, out_sems):
    cid = lax.axis_index("c")
    base = cid * _ROWS_PER_C

    def start_in(i):
        return pltpu.async_copy(
            x_hbm.at[pl.ds(base + i * _CH, _CH)], bufs[i % _NBUF],
            in_sems[i % _NBUF])

    def start_out(i):
        return pltpu.async_copy(
            bufs[i % _NBUF], o_hbm.at[pl.ds(base + i * _CH, _CH)],
            out_sems[i % _NBUF])

    in_copies = [None] * _NBUF
    out_copies = [None] * _NBUF
    for i in range(_PD):
        in_copies[i % _NBUF] = start_in(i)
    for i in range(_N_CH):
        b = i % _NBUF
        pf = i + _PD
        if pf < _N_CH:
            pb = pf % _NBUF
            if pf - _NBUF >= 0:
                out_copies[pb].wait()
            in_copies[pb] = start_in(pf)
        in_copies[b].wait()
        out_copies[b] = start_out(i)
    for c in out_copies:
        if c is not None:
            c.wait()


def kernel(x, heatmap0):
    del heatmap0
    b, c, h, w = x.shape
    x2 = x.reshape(_ROWS, _W)
    mesh = plsc.ScalarSubcoreMesh(axis_name="c", num_cores=_NC)
    run = functools.partial(
        pl.kernel,
        out_type=jax.ShapeDtypeStruct((_ROWS, _W), x.dtype),
        mesh=mesh,
        scratch_types=[
            [pltpu.VMEM_SHARED((_CH, _W), jnp.float32)] * _NBUF,
            [pltpu.SemaphoreType.DMA] * _NBUF,
            [pltpu.SemaphoreType.DMA] * _NBUF,
        ],
    )(_sc_copy)
    out = run(x2)
    return (out.reshape(b, c, h, w), jnp.float32(_THRESHOLD))


# speedup vs baseline: 1.0665x; 1.0665x over previous
import functools

import jax
import jax.numpy as jnp
from jax import lax
from jax.experimental import pallas as pl
from jax.experimental.pallas import tpu as pltpu
from jax.experimental.pallas import tpu_sc as plsc

_THRESHOLD = 0.5

_NC = 2
_NS = 16
_NW = _NC * _NS

_ROWS = 2 * 192 * 384
_W = 384
_ROWS_PER_W = _ROWS // _NW
_CH = 64
_N_CH = _ROWS_PER_W // _CH
_HALF = _N_CH // 2


def _sc_copy(x_hbm, o_hbm, abufs, bbufs, a_isems, a_osems, b_isems, b_osems):
    sid = lax.axis_index("s")
    wid = sid * _NC + lax.axis_index("c")
    base = wid * _ROWS_PER_W

    def src(chunk):
        return x_hbm.at[pl.ds(base + chunk * _CH, _CH)]

    def dst(chunk):
        return o_hbm.at[pl.ds(base + chunk * _CH, _CH)]

    def a_in(j):
        return pltpu.async_copy(src(2 * j), abufs[j % 2].at[sid], a_isems[j % 2])

    def a_out(j):
        return pltpu.async_copy(abufs[j % 2].at[sid], dst(2 * j), a_osems[j % 2])

    def b_in(j):
        return pltpu.async_copy(src(2 * j + 1), bbufs[j % 2], b_isems[j % 2])

    def b_out(j):
        return pltpu.async_copy(bbufs[j % 2], dst(2 * j + 1), b_osems[j % 2])

    A_in = [a_in(0), None]
    B_in = [b_in(0), None]
    A_out = [None, None]
    B_out = [None, None]
    for j in range(_HALF):
        b = j % 2
        nb = (j + 1) % 2
        if j + 1 < _HALF:
            if j >= 1:
                A_out[nb].wait()
            A_in[nb] = a_in(j + 1)
            if j >= 1:
                B_out[nb].wait()
            B_in[nb] = b_in(j + 1)
        A_in[b].wait()
        A_out[b] = a_out(j)
        B_in[b].wait()
        B_out[b] = b_out(j)
    for c in A_out + B_out:
        if c is not None:
            c.wait()


def kernel(x, heatmap0):
    del heatmap0
    b, c, h, w = x.shape
    x2 = x.reshape(_ROWS, _W)
    mesh = plsc.VectorSubcoreMesh(core_axis_name="c", subcore_axis_name="s")
    run = functools.partial(
        pl.kernel,
        out_type=jax.ShapeDtypeStruct((_ROWS, _W), x.dtype),
        mesh=mesh,
        scratch_types=[
            [pltpu.VMEM_SHARED((_NS, _CH, _W), jnp.float32)] * 2,
            [pltpu.VMEM((_CH, _W), jnp.float32)] * 2,
            [pltpu.SemaphoreType.DMA] * 2,
            [pltpu.SemaphoreType.DMA] * 2,
            [pltpu.SemaphoreType.DMA] * 2,
            [pltpu.SemaphoreType.DMA] * 2,
        ],
    )(_sc_copy)
    out = run(x2)
    return (out.reshape(b, c, h, w), jnp.float32(_THRESHOLD))


# speedup vs baseline: 1.3323x; 1.2492x over previous
import jax
import jax.numpy as jnp
from jax.experimental import pallas as pl
from jax.experimental.pallas import tpu as pltpu

_THRESHOLD = 0.5

_ROWS = 2 * 192 * 384
_W = 384
_BUF_ROWS = 8320

_CHUNKS = [1024, 2048, 4096] + [8320] * 16 + [4096, 2048, 1024]
assert sum(_CHUNKS) == _ROWS
_OFFS = [sum(_CHUNKS[:i]) for i in range(len(_CHUNKS))]
_NBUF = 4
_PD = 2


def _copy_ring(x_ref, o_ref, b0, b1, b2, b3, si0, si1, si2, si3,
               so0, so1, so2, so3):
    bufs = (b0, b1, b2, b3)
    in_sems = (si0, si1, si2, si3)
    out_sems = (so0, so1, so2, so3)
    n = len(_CHUNKS)

    def start_in(i):
        sz = _CHUNKS[i]
        return pltpu.async_copy(
            x_ref.at[pl.ds(_OFFS[i], sz)], bufs[i % _NBUF].at[pl.ds(0, sz)],
            in_sems[i % _NBUF])

    def start_out(i):
        sz = _CHUNKS[i]
        return pltpu.async_copy(
            bufs[i % _NBUF].at[pl.ds(0, sz)], o_ref.at[pl.ds(_OFFS[i], sz)],
            out_sems[i % _NBUF])

    in_copies = [None] * _NBUF
    out_copies = [None] * _NBUF
    for i in range(_PD):
        in_copies[i % _NBUF] = start_in(i)
    for i in range(n):
        b = i % _NBUF
        pf = i + _PD
        if pf < n:
            pb = pf % _NBUF
            if pf - _NBUF >= 0:
                out_copies[pb].wait()
            in_copies[pb] = start_in(pf)
        in_copies[b].wait()
        out_copies[b] = start_out(i)
    for c in out_copies:
        if c is not None:
            c.wait()


def kernel(x, heatmap0):
    del heatmap0
    b, c, h, w = x.shape
    x2 = x.reshape(_ROWS, _W)
    out = pl.pallas_call(
        _copy_ring,
        in_specs=[pl.BlockSpec(memory_space=pl.ANY)],
        out_specs=pl.BlockSpec(memory_space=pl.ANY),
        out_shape=jax.ShapeDtypeStruct((_ROWS, _W), x.dtype),
        scratch_shapes=(
            [pltpu.VMEM((_BUF_ROWS, _W), jnp.float32)] * _NBUF
            + [pltpu.SemaphoreType.DMA] * (2 * _NBUF)
        ),
    )(x2)
    return (out.reshape(b, c, h, w), jnp.float32(_THRESHOLD))
